# baseline (speedup 1.0000x reference)
import functools

import jax
import jax.numpy as jnp
from jax import lax
from jax.experimental import pallas as pl
from jax.experimental.pallas import tpu as pltpu
from jax.experimental.pallas import tpu_sc as plsc

N = 10000
E = 320000
D = 128

NC = 2
NS = 16
NW = NC * NS

SEGS = 5
SEG = E // SEGS

RPT = 624
TAIL = N - NS * RPT
GC = 80


def _leaky(z):
    return jnp.where(z > 0, z, 0.01 * z)


EPW = E // NW
NG = EPW // GC


def _gather_body(x0_hbm, src_hbm, xj_hbm, x0_sh, idx0, idx1, rows0, rows1,
                 isem0, isem1, gsem, wsem0, wsem1):
    c = lax.axis_index("c")
    s = lax.axis_index("s")
    wid = s * NC + c
    wb = wid * EPW
    pltpu.sync_copy(x0_hbm.at[pl.ds(s * RPT, RPT)],
                    x0_sh.at[pl.ds(s * RPT, RPT)])
    @pl.when(s == NS - 1)
    def _():
        pltpu.sync_copy(x0_hbm.at[pl.ds(NS * RPT, TAIL)],
                        x0_sh.at[pl.ds(NS * RPT, TAIL)])
    plsc.subcore_barrier()

    def src_slice(o):
        return src_hbm.at[pl.ds(pl.multiple_of(wb + o * GC, GC), GC)]

    def xj_slice(o):
        return xj_hbm.at[pl.ds(pl.multiple_of(wb + o * GC, GC), GC)]

    def stage(o, i, ibuf, rbuf, isem, wsem, first):
        pltpu.make_async_copy(src_slice(o), ibuf, isem).wait()
        @pl.when(jnp.logical_not(first))
        def _():
            pltpu.make_async_copy(rbuf, xj_slice(o - 2), wsem).wait()
        pltpu.async_copy(x0_sh.at[ibuf], rbuf, gsem).wait()
        pltpu.async_copy(rbuf, xj_slice(o), wsem)

    pltpu.async_copy(src_slice(0), idx0, isem0)

    def pair(i, carry):
        o = i * 2
        pltpu.async_copy(src_slice(o + 1), idx1, isem1)
        stage(o, i, idx0, rows0, isem0, wsem0, i == 0)
        pltpu.async_copy(src_slice(o + 2), idx0, isem0)
        stage(o + 1, i, idx1, rows1, isem1, wsem1, i == 0)
        return carry

    lax.fori_loop(0, NG // 2, pair, 0)
    stage(NG - 1, 0, idx0, rows0, isem0, wsem0, False)
    pltpu.make_async_copy(rows1, xj_slice(NG - 2), wsem1).wait()
    pltpu.make_async_copy(rows0, xj_slice(NG - 1), wsem0).wait()


def _sc_gather(x0, src):
    mesh = plsc.VectorSubcoreMesh(core_axis_name="c", subcore_axis_name="s",
                                  num_cores=NC, num_subcores=NS)
    return pl.kernel(
        _gather_body,
        out_type=jax.ShapeDtypeStruct((E, D), jnp.float32),
        mesh=mesh,
        scratch_types=[
            pltpu.VMEM_SHARED((N, D), jnp.float32),
            pltpu.VMEM((GC,), jnp.int32),
            pltpu.VMEM((GC,), jnp.int32),
            pltpu.VMEM((GC, D), jnp.float32),
            pltpu.VMEM((GC, D), jnp.float32),
            pltpu.SemaphoreType.DMA,
            pltpu.SemaphoreType.DMA,
            pltpu.SemaphoreType.DMA,
            pltpu.SemaphoreType.DMA,
            pltpu.SemaphoreType.DMA,
        ],
    )(x0, src)


OC = 160
NSUB = OC // GC
EPT = SEG // NS
NOUT = EPT // OC


def _scatter_body(msg_hbm, dst4d_hbm, prev_hbm, aggr_hbm, acc_sh,
                  idx_all, rows0, rows1, isem, psem, vsem0, vsem1, ssem):
    c = lax.axis_index("c")
    s = lax.axis_index("s")
    tb = s * EPT

    pltpu.async_copy(dst4d_hbm.at[s], idx_all, isem)

    for k in range(2):
        col0 = (c * 2 + k) * 128

        def msg_slice(o):
            return msg_hbm.at[pl.ds(tb + o * OC, OC), pl.ds(col0, 128)]

        def wait_load(o, rbuf, vsem):
            pltpu.make_async_copy(msg_slice(o), rbuf, vsem).wait()

        def scat(o, buf):
            ds_ = []
            for j in range(NSUB):
                ds_.append(pltpu.async_copy(
                    buf.at[pl.ds(j * GC, GC)],
                    acc_sh.at[idx_all.at[o, j]], ssem, add=True))
            for d in ds_:
                d.wait()

        pltpu.async_copy(prev_hbm.at[pl.ds(s * RPT, RPT), pl.ds(col0, 128)],
                         acc_sh.at[pl.ds(s * RPT, RPT)], psem)
        @pl.when(s == NS - 1)
        def _():
            pltpu.async_copy(prev_hbm.at[pl.ds(NS * RPT, TAIL),
                                         pl.ds(col0, 128)],
                             acc_sh.at[pl.ds(NS * RPT, TAIL)], psem)
        pltpu.async_copy(msg_slice(0), rows0, vsem0)
        pltpu.make_async_copy(
            prev_hbm.at[pl.ds(s * RPT, RPT), pl.ds(col0, 128)],
            acc_sh.at[pl.ds(s * RPT, RPT)], psem).wait()
        @pl.when(s == NS - 1)
        def _():
            pltpu.make_async_copy(
                prev_hbm.at[pl.ds(NS * RPT, TAIL), pl.ds(col0, 128)],
                acc_sh.at[pl.ds(NS * RPT, TAIL)], psem).wait()
        if k == 0:
            pltpu.make_async_copy(dst4d_hbm.at[s], idx_all, isem).wait()
        plsc.subcore_barrier()

        def pair(i, carry):
            o = i * 2
            pltpu.async_copy(msg_slice(o + 1), rows1, vsem1)
            wait_load(o, rows0, vsem0)
            scat(o, rows0)
            pltpu.async_copy(msg_slice(o + 2), rows0, vsem0)
            wait_load(o + 1, rows1, vsem1)
            scat(o + 1, rows1)
            return carry

        lax.fori_loop(0, NOUT // 2, pair, 0)
        wait_load(NOUT - 1, rows0, vsem0)
        scat(NOUT - 1, rows0)
        plsc.subcore_barrier()
        pltpu.sync_copy(acc_sh.at[pl.ds(s * RPT, RPT)],
                        aggr_hbm.at[pl.ds(s * RPT, RPT), pl.ds(col0, 128)])
        @pl.when(s == NS - 1)
        def _():
            pltpu.sync_copy(acc_sh.at[pl.ds(NS * RPT, TAIL)],
                            aggr_hbm.at[pl.ds(NS * RPT, TAIL),
                                        pl.ds(col0, 128)])


def _sc_scatter(msg_seg, dst_seg, prev):
    mesh = plsc.VectorSubcoreMesh(core_axis_name="c", subcore_axis_name="s",
                                  num_cores=NC, num_subcores=NS)
    dst4d = dst_seg.reshape(NS, NOUT, NSUB, GC)
    return pl.kernel(
        _scatter_body,
        out_type=jax.ShapeDtypeStruct((N, 512), jnp.float32),
        mesh=mesh,
        scratch_types=[
            pltpu.VMEM_SHARED((N, 128), jnp.float32),
            pltpu.VMEM((NOUT, NSUB, GC), jnp.int32),
            pltpu.VMEM((OC, 128), jnp.float32),
            pltpu.VMEM((OC, 128), jnp.float32),
            pltpu.SemaphoreType.DMA,
            pltpu.SemaphoreType.DMA,
            pltpu.SemaphoreType.DMA,
            pltpu.SemaphoreType.DMA,
            pltpu.SemaphoreType.DMA,
        ],
    )(msg_seg, dst4d, prev)


def _edge_mlp_body(xj, ea, W1, b1, W2, b2, out):
    t = xj[...] * ea[...]
    h = _leaky(jnp.dot(t, W1[...], preferred_element_type=jnp.float32)
               + b1[...])
    out[...] = _leaky(jnp.dot(h, W2[...], preferred_element_type=jnp.float32)
                      + b2[...])


def _edge_mlp(xj, ea, W1, b1, W2, b2, sg):
    BE = 1600
    grid = (SEG // BE,)
    off = sg * (SEG // BE)
    return pl.pallas_call(
        _edge_mlp_body,
        grid=grid,
        in_specs=[
            pl.BlockSpec((BE, D), lambda i: (off + i, 0)),
            pl.BlockSpec((BE, D), lambda i: (off + i, 0)),
            pl.BlockSpec((D, 256), lambda i: (0, 0)),
            pl.BlockSpec((1, 256), lambda i: (0, 0)),
            pl.BlockSpec((256, 512), lambda i: (0, 0)),
            pl.BlockSpec((1, 512), lambda i: (0, 0)),
        ],
        out_specs=pl.BlockSpec((BE, 512), lambda i: (i, 0)),
        out_shape=jax.ShapeDtypeStruct((SEG, 512), jnp.float32),
    )(xj, ea, W1, b1, W2, b2)


def _node_mlp_body(x0, aggr, W3a, W3b, b3, W4, b4s, out):
    bf = jnp.bfloat16
    u = _leaky(jnp.dot(x0[...].astype(bf), W3a[...].astype(bf),
                       preferred_element_type=jnp.float32)
               + jnp.dot(aggr[...].astype(bf), W3b[...].astype(bf),
                         preferred_element_type=jnp.float32)
               + b3[...])
    y = jnp.dot(u.astype(bf), W4[...].astype(bf),
                preferred_element_type=jnp.float32)
    out[...] = (y + b4s[...] + x0[...]) * 0.5


def _node_mlp(x0, aggr, W3a_s, W3b_s, b3, W4_s, b4s):
    BN = 1000
    grid = (N // BN,)
    return pl.pallas_call(
        _node_mlp_body,
        grid=grid,
        in_specs=[
            pl.BlockSpec((BN, D), lambda i: (i, 0)),
            pl.BlockSpec((BN, 512), lambda i: (i, 0)),
            pl.BlockSpec((D, 1024), lambda i: (0, 0)),
            pl.BlockSpec((512, 1024), lambda i: (0, 0)),
            pl.BlockSpec((1, 1024), lambda i: (0, 0)),
            pl.BlockSpec((1024, D), lambda i: (0, 0)),
            pl.BlockSpec((1, D), lambda i: (0, 0)),
        ],
        out_specs=pl.BlockSpec((BN, D), lambda i: (i, 0)),
        out_shape=jax.ShapeDtypeStruct((N, D), jnp.float32),
    )(x0, aggr, W3a_s, W3b_s, b3, W4_s, b4s)


def kernel(x0, edge_index, edge_attr, W1, b1, W2, b2, W3, b3, W4, b4,
           bn_g, bn_b, bn_m, bn_v):
    src = edge_index[0].astype(jnp.int32)
    dst = edge_index[1].astype(jnp.int32)

    b1r = b1.reshape(1, 256)
    b2r = b2.reshape(1, 512)

    xj = _sc_gather(x0, src)
    aggr = jnp.zeros((N, 512), jnp.float32)
    for sg in range(SEGS):
        lo = sg * SEG
        msg = _edge_mlp(xj, edge_attr, W1, b1r, W2, b2r, sg)
        aggr = _sc_scatter(msg, lax.slice(dst, (lo,), (lo + SEG,)), aggr)

    scale = bn_g / jnp.sqrt(bn_v + 1e-5)
    shift = (b4 - bn_m) * scale + bn_b
    W4_s = W4 * scale[None, :]
    b4s = shift.reshape(1, D)
    out = _node_mlp(x0, aggr, W3[:D], W3[D:], b3.reshape(1, 1024), W4_s, b4s)
    return out


# speedup vs baseline: 1.0269x; 1.0269x over previous
import functools

import jax
import jax.numpy as jnp
from jax import lax
from jax.experimental import pallas as pl
from jax.experimental.pallas import tpu as pltpu
from jax.experimental.pallas import tpu_sc as plsc

N = 10000
E = 320000
D = 128

NC = 2
NS = 16
NW = NC * NS

SEGS = 5
SEG = E // SEGS

RPT = 624
TAIL = N - NS * RPT
GC = 80


def _leaky(z):
    return jnp.where(z > 0, z, 0.01 * z)


EPW = E // NW
NG = EPW // GC


def _gather_body(x0_hbm, src_hbm, xj_hbm, x0_sh, idx0, idx1, rows0, rows1,
                 isem0, isem1, gsem, wsem0, wsem1):
    c = lax.axis_index("c")
    s = lax.axis_index("s")
    wid = s * NC + c
    wb = wid * EPW
    pltpu.sync_copy(x0_hbm.at[pl.ds(s * RPT, RPT)],
                    x0_sh.at[pl.ds(s * RPT, RPT)])
    @pl.when(s == NS - 1)
    def _():
        pltpu.sync_copy(x0_hbm.at[pl.ds(NS * RPT, TAIL)],
                        x0_sh.at[pl.ds(NS * RPT, TAIL)])
    plsc.subcore_barrier()

    def src_slice(o):
        return src_hbm.at[pl.ds(pl.multiple_of(wb + o * GC, GC), GC)]

    def xj_slice(o):
        return xj_hbm.at[pl.ds(pl.multiple_of(wb + o * GC, GC), GC)]

    def stage(o, i, ibuf, rbuf, isem, wsem, first):
        pltpu.make_async_copy(src_slice(o), ibuf, isem).wait()
        @pl.when(jnp.logical_not(first))
        def _():
            pltpu.make_async_copy(rbuf, xj_slice(o - 2), wsem).wait()
        pltpu.async_copy(x0_sh.at[ibuf], rbuf, gsem).wait()
        pltpu.async_copy(rbuf, xj_slice(o), wsem)

    pltpu.async_copy(src_slice(0), idx0, isem0)

    def pair(i, carry):
        o = i * 2
        pltpu.async_copy(src_slice(o + 1), idx1, isem1)
        stage(o, i, idx0, rows0, isem0, wsem0, i == 0)
        pltpu.async_copy(src_slice(o + 2), idx0, isem0)
        stage(o + 1, i, idx1, rows1, isem1, wsem1, i == 0)
        return carry

    lax.fori_loop(0, NG // 2, pair, 0)
    stage(NG - 1, 0, idx0, rows0, isem0, wsem0, False)
    pltpu.make_async_copy(rows1, xj_slice(NG - 2), wsem1).wait()
    pltpu.make_async_copy(rows0, xj_slice(NG - 1), wsem0).wait()


def _sc_gather(x0, src):
    mesh = plsc.VectorSubcoreMesh(core_axis_name="c", subcore_axis_name="s",
                                  num_cores=NC, num_subcores=NS)
    return pl.kernel(
        _gather_body,
        out_type=jax.ShapeDtypeStruct((E, D), jnp.float32),
        mesh=mesh,
        scratch_types=[
            pltpu.VMEM_SHARED((N, D), jnp.float32),
            pltpu.VMEM((GC,), jnp.int32),
            pltpu.VMEM((GC,), jnp.int32),
            pltpu.VMEM((GC, D), jnp.float32),
            pltpu.VMEM((GC, D), jnp.float32),
            pltpu.SemaphoreType.DMA,
            pltpu.SemaphoreType.DMA,
            pltpu.SemaphoreType.DMA,
            pltpu.SemaphoreType.DMA,
            pltpu.SemaphoreType.DMA,
        ],
    )(x0, src)


OC = 160
NSUB = OC // GC
EPT = SEG // NS
NOUT = EPT // OC


def _scatter_body(msg_hbm, dst4d_hbm, prev_hbm, aggr_hbm, acc_sh,
                  idx_all, rows0, rows1, isem, psem, vsem0, vsem1, ssem):
    c = lax.axis_index("c")
    s = lax.axis_index("s")
    tb = s * EPT

    pltpu.async_copy(dst4d_hbm.at[s], idx_all, isem)

    for k in range(2):
        col0 = (c * 2 + k) * 128

        def msg_slice(o):
            return msg_hbm.at[pl.ds(tb + o * OC, OC), pl.ds(col0, 128)]

        def wait_load(o, rbuf, vsem):
            pltpu.make_async_copy(msg_slice(o), rbuf, vsem).wait()

        def scat(o, buf):
            ds_ = []
            for j in range(NSUB):
                ds_.append(pltpu.async_copy(
                    buf.at[pl.ds(j * GC, GC)],
                    acc_sh.at[idx_all.at[o, j]], ssem, add=True))
            for d in ds_:
                d.wait()

        pltpu.async_copy(prev_hbm.at[pl.ds(s * RPT, RPT), pl.ds(col0, 128)],
                         acc_sh.at[pl.ds(s * RPT, RPT)], psem)
        @pl.when(s == NS - 1)
        def _():
            pltpu.async_copy(prev_hbm.at[pl.ds(NS * RPT, TAIL),
                                         pl.ds(col0, 128)],
                             acc_sh.at[pl.ds(NS * RPT, TAIL)], psem)
        pltpu.async_copy(msg_slice(0), rows0, vsem0)
        pltpu.make_async_copy(
            prev_hbm.at[pl.ds(s * RPT, RPT), pl.ds(col0, 128)],
            acc_sh.at[pl.ds(s * RPT, RPT)], psem).wait()
        @pl.when(s == NS - 1)
        def _():
            pltpu.make_async_copy(
                prev_hbm.at[pl.ds(NS * RPT, TAIL), pl.ds(col0, 128)],
                acc_sh.at[pl.ds(NS * RPT, TAIL)], psem).wait()
        if k == 0:
            pltpu.make_async_copy(dst4d_hbm.at[s], idx_all, isem).wait()
        plsc.subcore_barrier()

        def pair(i, carry):
            o = i * 2
            pltpu.async_copy(msg_slice(o + 1), rows1, vsem1)
            wait_load(o, rows0, vsem0)
            scat(o, rows0)
            pltpu.async_copy(msg_slice(o + 2), rows0, vsem0)
            wait_load(o + 1, rows1, vsem1)
            scat(o + 1, rows1)
            return carry

        lax.fori_loop(0, NOUT // 2, pair, 0)
        wait_load(NOUT - 1, rows0, vsem0)
        scat(NOUT - 1, rows0)
        plsc.subcore_barrier()
        pltpu.sync_copy(acc_sh.at[pl.ds(s * RPT, RPT)],
                        aggr_hbm.at[pl.ds(s * RPT, RPT), pl.ds(col0, 128)])
        @pl.when(s == NS - 1)
        def _():
            pltpu.sync_copy(acc_sh.at[pl.ds(NS * RPT, TAIL)],
                            aggr_hbm.at[pl.ds(NS * RPT, TAIL),
                                        pl.ds(col0, 128)])


def _sc_scatter(msg_seg, dst_seg, prev):
    mesh = plsc.VectorSubcoreMesh(core_axis_name="c", subcore_axis_name="s",
                                  num_cores=NC, num_subcores=NS)
    dst4d = dst_seg.reshape(NS, NOUT, NSUB, GC)
    return pl.kernel(
        _scatter_body,
        out_type=jax.ShapeDtypeStruct((N, 512), jnp.float32),
        mesh=mesh,
        scratch_types=[
            pltpu.VMEM_SHARED((N, 128), jnp.float32),
            pltpu.VMEM((NOUT, NSUB, GC), jnp.int32),
            pltpu.VMEM((OC, 128), jnp.float32),
            pltpu.VMEM((OC, 128), jnp.float32),
            pltpu.SemaphoreType.DMA,
            pltpu.SemaphoreType.DMA,
            pltpu.SemaphoreType.DMA,
            pltpu.SemaphoreType.DMA,
            pltpu.SemaphoreType.DMA,
        ],
    )(msg_seg, dst4d, prev)


def _edge_mlp_body(xj, ea, W1, b1, W2, b2, out):
    t = xj[...] * ea[...]
    h = _leaky(jnp.dot(t, W1[...], preferred_element_type=jnp.float32)
               + b1[...])
    out[...] = _leaky(jnp.dot(h, W2[...], preferred_element_type=jnp.float32)
                      + b2[...])


def _edge_mlp(xj, ea, W1, b1, W2, b2, sg):
    BE = 800
    grid = (SEG // BE,)
    off = sg * (SEG // BE)
    return pl.pallas_call(
        _edge_mlp_body,
        grid=grid,
        in_specs=[
            pl.BlockSpec((BE, D), lambda i: (off + i, 0)),
            pl.BlockSpec((BE, D), lambda i: (off + i, 0)),
            pl.BlockSpec((D, 256), lambda i: (0, 0)),
            pl.BlockSpec((1, 256), lambda i: (0, 0)),
            pl.BlockSpec((256, 512), lambda i: (0, 0)),
            pl.BlockSpec((1, 512), lambda i: (0, 0)),
        ],
        out_specs=pl.BlockSpec((BE, 512), lambda i: (i, 0)),
        out_shape=jax.ShapeDtypeStruct((SEG, 512), jnp.float32),
    )(xj, ea, W1, b1, W2, b2)


def _node_mlp_body(x0, aggr, W3a, W3b, b3, W4, b4s, out):
    bf = jnp.bfloat16
    u = _leaky(jnp.dot(x0[...].astype(bf), W3a[...].astype(bf),
                       preferred_element_type=jnp.float32)
               + jnp.dot(aggr[...].astype(bf), W3b[...].astype(bf),
                         preferred_element_type=jnp.float32)
               + b3[...])
    y = jnp.dot(u.astype(bf), W4[...].astype(bf),
                preferred_element_type=jnp.float32)
    out[...] = (y + b4s[...] + x0[...]) * 0.5


def _node_mlp(x0, aggr, W3a_s, W3b_s, b3, W4_s, b4s):
    BN = 1000
    grid = (N // BN,)
    return pl.pallas_call(
        _node_mlp_body,
        grid=grid,
        in_specs=[
            pl.BlockSpec((BN, D), lambda i: (i, 0)),
            pl.BlockSpec((BN, 512), lambda i: (i, 0)),
            pl.BlockSpec((D, 1024), lambda i: (0, 0)),
            pl.BlockSpec((512, 1024), lambda i: (0, 0)),
            pl.BlockSpec((1, 1024), lambda i: (0, 0)),
            pl.BlockSpec((1024, D), lambda i: (0, 0)),
            pl.BlockSpec((1, D), lambda i: (0, 0)),
        ],
        out_specs=pl.BlockSpec((BN, D), lambda i: (i, 0)),
        out_shape=jax.ShapeDtypeStruct((N, D), jnp.float32),
    )(x0, aggr, W3a_s, W3b_s, b3, W4_s, b4s)


def kernel(x0, edge_index, edge_attr, W1, b1, W2, b2, W3, b3, W4, b4,
           bn_g, bn_b, bn_m, bn_v):
    src = edge_index[0].astype(jnp.int32)
    dst = edge_index[1].astype(jnp.int32)

    b1r = b1.reshape(1, 256)
    b2r = b2.reshape(1, 512)

    xj = _sc_gather(x0, src)
    aggr = jnp.zeros((N, 512), jnp.float32)
    for sg in range(SEGS):
        lo = sg * SEG
        msg = _edge_mlp(xj, edge_attr, W1, b1r, W2, b2r, sg)
        aggr = _sc_scatter(msg, lax.slice(dst, (lo,), (lo + SEG,)), aggr)

    scale = bn_g / jnp.sqrt(bn_v + 1e-5)
    shift = (b4 - bn_m) * scale + bn_b
    W4_s = W4 * scale[None, :]
    b4s = shift.reshape(1, D)
    out = _node_mlp(x0, aggr, W3[:D], W3[D:], b3.reshape(1, 1024), W4_s, b4s)
    return out


# speedup vs baseline: 1.0545x; 1.0268x over previous
import functools

import jax
import jax.numpy as jnp
from jax import lax
from jax.experimental import pallas as pl
from jax.experimental.pallas import tpu as pltpu
from jax.experimental.pallas import tpu_sc as plsc

N = 10000
E = 320000
D = 128

NC = 2
NS = 16
NW = NC * NS

GC = 80
OC = 160
NSUB = OC // GC

SEG_SIZES = (38400, 76800, 76800, 76800, 51200)
G0_SIZE = SEG_SIZES[0]

RPT = 624
TAIL = N - NS * RPT


def _leaky(z):
    return jnp.where(z > 0, z, 0.01 * z)


@functools.lru_cache(maxsize=None)
def _make_gather(nedge):
    epw = nedge // NW
    ng = epw // GC
    assert epw % GC == 0

    def body(x0_hbm, src_hbm, xj_hbm, x0_sh, idx0, idx1, rows0, rows1,
             isem0, isem1, gsem, wsem0, wsem1):
        c = lax.axis_index("c")
        s = lax.axis_index("s")
        wid = s * NC + c
        wb = wid * epw
        pltpu.sync_copy(x0_hbm.at[pl.ds(s * RPT, RPT)],
                        x0_sh.at[pl.ds(s * RPT, RPT)])
        @pl.when(s == NS - 1)
        def _():
            pltpu.sync_copy(x0_hbm.at[pl.ds(NS * RPT, TAIL)],
                            x0_sh.at[pl.ds(NS * RPT, TAIL)])
        plsc.subcore_barrier()

        def src_slice(o):
            return src_hbm.at[pl.ds(pl.multiple_of(wb + o * GC, GC), GC)]

        def xj_slice(o):
            return xj_hbm.at[pl.ds(pl.multiple_of(wb + o * GC, GC), GC)]

        def stage(o, ibuf, rbuf, isem, wsem, first):
            pltpu.make_async_copy(src_slice(o), ibuf, isem).wait()
            @pl.when(jnp.logical_not(first))
            def _():
                pltpu.make_async_copy(rbuf, xj_slice(o - 2), wsem).wait()
            pltpu.async_copy(x0_sh.at[ibuf], rbuf, gsem).wait()
            pltpu.async_copy(rbuf, xj_slice(o), wsem)

        pltpu.async_copy(src_slice(0), idx0, isem0)

        def pair(i, carry):
            o = i * 2
            pltpu.async_copy(src_slice(o + 1), idx1, isem1)
            stage(o, idx0, rows0, isem0, wsem0, i == 0)
            @pl.when(o + 2 < ng)
            def _():
                pltpu.async_copy(src_slice(o + 2), idx0, isem0)
            stage(o + 1, idx1, rows1, isem1, wsem1, i == 0)
            return carry

        lax.fori_loop(0, ng // 2, pair, 0)
        if ng % 2:
            stage(ng - 1, idx0, rows0, isem0, wsem0, False)
            pltpu.make_async_copy(rows1, xj_slice(ng - 2), wsem1).wait()
            pltpu.make_async_copy(rows0, xj_slice(ng - 1), wsem0).wait()
        else:
            pltpu.make_async_copy(rows0, xj_slice(ng - 2), wsem0).wait()
            pltpu.make_async_copy(rows1, xj_slice(ng - 1), wsem1).wait()

    mesh = plsc.VectorSubcoreMesh(core_axis_name="c", subcore_axis_name="s",
                                  num_cores=NC, num_subcores=NS)
    return pl.kernel(
        body,
        out_type=jax.ShapeDtypeStruct((nedge, D), jnp.float32),
        mesh=mesh,
        scratch_types=[
            pltpu.VMEM_SHARED((N, D), jnp.float32),
            pltpu.VMEM((GC,), jnp.int32),
            pltpu.VMEM((GC,), jnp.int32),
            pltpu.VMEM((GC, D), jnp.float32),
            pltpu.VMEM((GC, D), jnp.float32),
            pltpu.SemaphoreType.DMA,
            pltpu.SemaphoreType.DMA,
            pltpu.SemaphoreType.DMA,
            pltpu.SemaphoreType.DMA,
            pltpu.SemaphoreType.DMA,
        ],
    )


@functools.lru_cache(maxsize=None)
def _make_scatter(nedge):
    ept = nedge // NS
    nout = ept // OC
    assert ept % OC == 0

    def body(msg_hbm, dst4d_hbm, prev_hbm, aggr_hbm, acc_sh,
             idx_all, rows0, rows1, isem, psem, vsem0, vsem1, ssem):
        c = lax.axis_index("c")
        s = lax.axis_index("s")
        tb = s * ept

        pltpu.async_copy(dst4d_hbm.at[s], idx_all, isem)

        for k in range(2):
            col0 = (c * 2 + k) * 128

            def msg_slice(o):
                return msg_hbm.at[pl.ds(tb + o * OC, OC), pl.ds(col0, 128)]

            def wait_load(o, rbuf, vsem):
                pltpu.make_async_copy(msg_slice(o), rbuf, vsem).wait()

            def scat(o, buf):
                ds_ = []
                for j in range(NSUB):
                    ds_.append(pltpu.async_copy(
                        buf.at[pl.ds(j * GC, GC)],
                        acc_sh.at[idx_all.at[o, j]], ssem, add=True))
                for d in ds_:
                    d.wait()

            pltpu.async_copy(
                prev_hbm.at[pl.ds(s * RPT, RPT), pl.ds(col0, 128)],
                acc_sh.at[pl.ds(s * RPT, RPT)], psem)
            @pl.when(s == NS - 1)
            def _():
                pltpu.async_copy(
                    prev_hbm.at[pl.ds(NS * RPT, TAIL), pl.ds(col0, 128)],
                    acc_sh.at[pl.ds(NS * RPT, TAIL)], psem)
            pltpu.async_copy(msg_slice(0), rows0, vsem0)
            pltpu.make_async_copy(
                prev_hbm.at[pl.ds(s * RPT, RPT), pl.ds(col0, 128)],
                acc_sh.at[pl.ds(s * RPT, RPT)], psem).wait()
            @pl.when(s == NS - 1)
            def _():
                pltpu.make_async_copy(
                    prev_hbm.at[pl.ds(NS * RPT, TAIL), pl.ds(col0, 128)],
                    acc_sh.at[pl.ds(NS * RPT, TAIL)], psem).wait()
            if k == 0:
                pltpu.make_async_copy(dst4d_hbm.at[s], idx_all, isem).wait()
            plsc.subcore_barrier()

            def pair(i, carry):
                o = i * 2
                pltpu.async_copy(msg_slice(o + 1), rows1, vsem1)
                wait_load(o, rows0, vsem0)
                scat(o, rows0)
                @pl.when(o + 2 < nout)
                def _():
                    pltpu.async_copy(msg_slice(o + 2), rows0, vsem0)
                wait_load(o + 1, rows1, vsem1)
                scat(o + 1, rows1)
                return carry

            lax.fori_loop(0, nout // 2, pair, 0)
            if nout % 2:
                wait_load(nout - 1, rows0, vsem0)
                scat(nout - 1, rows0)
            plsc.subcore_barrier()
            pltpu.sync_copy(acc_sh.at[pl.ds(s * RPT, RPT)],
                            aggr_hbm.at[pl.ds(s * RPT, RPT),
                                        pl.ds(col0, 128)])
            @pl.when(s == NS - 1)
            def _():
                pltpu.sync_copy(acc_sh.at[pl.ds(NS * RPT, TAIL)],
                                aggr_hbm.at[pl.ds(NS * RPT, TAIL),
                                            pl.ds(col0, 128)])

    mesh = plsc.VectorSubcoreMesh(core_axis_name="c", subcore_axis_name="s",
                                  num_cores=NC, num_subcores=NS)
    kern = pl.kernel(
        body,
        out_type=jax.ShapeDtypeStruct((N, 512), jnp.float32),
        mesh=mesh,
        scratch_types=[
            pltpu.VMEM_SHARED((N, 128), jnp.float32),
            pltpu.VMEM((nout, NSUB, GC), jnp.int32),
            pltpu.VMEM((OC, 128), jnp.float32),
            pltpu.VMEM((OC, 128), jnp.float32),
            pltpu.SemaphoreType.DMA,
            pltpu.SemaphoreType.DMA,
            pltpu.SemaphoreType.DMA,
            pltpu.SemaphoreType.DMA,
            pltpu.SemaphoreType.DMA,
        ],
    )

    def run(msg_seg, dst_seg, prev):
        dst4d = dst_seg.reshape(NS, nout, NSUB, GC)
        return kern(msg_seg, dst4d, prev)

    return run


def _edge_mlp_body(xj, ea, W1, b1, W2, b2, out):
    t = xj[...] * ea[...]
    h = _leaky(jnp.dot(t, W1[...], preferred_element_type=jnp.float32)
               + b1[...])
    out[...] = _leaky(jnp.dot(h, W2[...], preferred_element_type=jnp.float32)
                      + b2[...])


def _edge_mlp(xj, xj_off, ea, ea_off, nedge, W1, b1, W2, b2):
    BE = 800
    grid = (nedge // BE,)
    xo = xj_off // BE
    eo = ea_off // BE
    return pl.pallas_call(
        _edge_mlp_body,
        grid=grid,
        in_specs=[
            pl.BlockSpec((BE, D), lambda i: (xo + i, 0)),
            pl.BlockSpec((BE, D), lambda i: (eo + i, 0)),
            pl.BlockSpec((D, 256), lambda i: (0, 0)),
            pl.BlockSpec((1, 256), lambda i: (0, 0)),
            pl.BlockSpec((256, 512), lambda i: (0, 0)),
            pl.BlockSpec((1, 512), lambda i: (0, 0)),
        ],
        out_specs=pl.BlockSpec((BE, 512), lambda i: (i, 0)),
        out_shape=jax.ShapeDtypeStruct((nedge, 512), jnp.float32),
    )(xj, ea, W1, b1, W2, b2)


def _node_mlp_body(x0, aggr, W3a, W3b, b3, W4, b4s, out):
    bf = jnp.bfloat16
    u = _leaky(jnp.dot(x0[...].astype(bf), W3a[...].astype(bf),
                       preferred_element_type=jnp.float32)
               + jnp.dot(aggr[...].astype(bf), W3b[...].astype(bf),
                         preferred_element_type=jnp.float32)
               + b3[...])
    y = jnp.dot(u.astype(bf), W4[...].astype(bf),
                preferred_element_type=jnp.float32)
    out[...] = (y + b4s[...] + x0[...]) * 0.5


def _node_mlp(x0, aggr, W3a_s, W3b_s, b3, W4_s, b4s):
    BN = 1000
    grid = (N // BN,)
    return pl.pallas_call(
        _node_mlp_body,
        grid=grid,
        in_specs=[
            pl.BlockSpec((BN, D), lambda i: (i, 0)),
            pl.BlockSpec((BN, 512), lambda i: (i, 0)),
            pl.BlockSpec((D, 1024), lambda i: (0, 0)),
            pl.BlockSpec((512, 1024), lambda i: (0, 0)),
            pl.BlockSpec((1, 1024), lambda i: (0, 0)),
            pl.BlockSpec((1024, D), lambda i: (0, 0)),
            pl.BlockSpec((1, D), lambda i: (0, 0)),
        ],
        out_specs=pl.BlockSpec((BN, D), lambda i: (i, 0)),
        out_shape=jax.ShapeDtypeStruct((N, D), jnp.float32),
    )(x0, aggr, W3a_s, W3b_s, b3, W4_s, b4s)


def kernel(x0, edge_index, edge_attr, W1, b1, W2, b2, W3, b3, W4, b4,
           bn_g, bn_b, bn_m, bn_v):
    src = edge_index[0].astype(jnp.int32)
    dst = edge_index[1].astype(jnp.int32)

    b1r = b1.reshape(1, 256)
    b2r = b2.reshape(1, 512)

    xj_a = _make_gather(G0_SIZE)(x0, lax.slice(src, (0,), (G0_SIZE,)))
    xj_b = _make_gather(E - G0_SIZE)(x0, lax.slice(src, (G0_SIZE,), (E,)))

    aggr = jnp.zeros((N, 512), jnp.float32)
    lo = 0
    for sg, sz in enumerate(SEG_SIZES):
        if sg == 0:
            msg = _edge_mlp(xj_a, 0, edge_attr, 0, sz, W1, b1r, W2, b2r)
        else:
            msg = _edge_mlp(xj_b, lo - G0_SIZE, edge_attr, lo, sz,
                            W1, b1r, W2, b2r)
        aggr = _make_scatter(sz)(
            msg, lax.slice(dst, (lo,), (lo + sz,)), aggr)
        lo += sz

    scale = bn_g / jnp.sqrt(bn_v + 1e-5)
    shift = (b4 - bn_m) * scale + bn_b
    W4_s = W4 * scale[None, :]
    b4s = shift.reshape(1, D)
    out = _node_mlp(x0, aggr, W3[:D], W3[D:], b3.reshape(1, 1024), W4_s, b4s)
    return out


# speedup vs baseline: 1.0635x; 1.0086x over previous
import functools

import jax
import jax.numpy as jnp
from jax import lax
from jax.experimental import pallas as pl
from jax.experimental.pallas import tpu as pltpu
from jax.experimental.pallas import tpu_sc as plsc

N = 10000
E = 320000
D = 128

NC = 2
NS = 16
NW = NC * NS

GC = 80
OC = 160
NSUB = OC // GC

SEG_SIZES = (51200, 89600, 89600, 89600)
G0_SIZE = SEG_SIZES[0]

RPT = 624
TAIL = N - NS * RPT


def _leaky(z):
    return jnp.where(z > 0, z, 0.01 * z)


@functools.lru_cache(maxsize=None)
def _make_gather(nedge):
    epw = nedge // NW
    ng = epw // GC
    assert epw % GC == 0

    def body(x0_hbm, src_hbm, xj_hbm, x0_sh, idx0, idx1, rows0, rows1,
             isem0, isem1, gsem, wsem0, wsem1):
        c = lax.axis_index("c")
        s = lax.axis_index("s")
        wid = s * NC + c
        wb = wid * epw
        pltpu.sync_copy(x0_hbm.at[pl.ds(s * RPT, RPT)],
                        x0_sh.at[pl.ds(s * RPT, RPT)])
        @pl.when(s == NS - 1)
        def _():
            pltpu.sync_copy(x0_hbm.at[pl.ds(NS * RPT, TAIL)],
                            x0_sh.at[pl.ds(NS * RPT, TAIL)])
        plsc.subcore_barrier()

        def src_slice(o):
            return src_hbm.at[pl.ds(pl.multiple_of(wb + o * GC, GC), GC)]

        def xj_slice(o):
            return xj_hbm.at[pl.ds(pl.multiple_of(wb + o * GC, GC), GC)]

        def stage(o, ibuf, rbuf, isem, wsem, first):
            pltpu.make_async_copy(src_slice(o), ibuf, isem).wait()
            @pl.when(jnp.logical_not(first))
            def _():
                pltpu.make_async_copy(rbuf, xj_slice(o - 2), wsem).wait()
            pltpu.async_copy(x0_sh.at[ibuf], rbuf, gsem).wait()
            pltpu.async_copy(rbuf, xj_slice(o), wsem)

        pltpu.async_copy(src_slice(0), idx0, isem0)

        def pair(i, carry):
            o = i * 2
            pltpu.async_copy(src_slice(o + 1), idx1, isem1)
            stage(o, idx0, rows0, isem0, wsem0, i == 0)
            @pl.when(o + 2 < ng)
            def _():
                pltpu.async_copy(src_slice(o + 2), idx0, isem0)
            stage(o + 1, idx1, rows1, isem1, wsem1, i == 0)
            return carry

        lax.fori_loop(0, ng // 2, pair, 0)
        if ng % 2:
            stage(ng - 1, idx0, rows0, isem0, wsem0, False)
            pltpu.make_async_copy(rows1, xj_slice(ng - 2), wsem1).wait()
            pltpu.make_async_copy(rows0, xj_slice(ng - 1), wsem0).wait()
        else:
            pltpu.make_async_copy(rows0, xj_slice(ng - 2), wsem0).wait()
            pltpu.make_async_copy(rows1, xj_slice(ng - 1), wsem1).wait()

    mesh = plsc.VectorSubcoreMesh(core_axis_name="c", subcore_axis_name="s",
                                  num_cores=NC, num_subcores=NS)
    return pl.kernel(
        body,
        out_type=jax.ShapeDtypeStruct((nedge, D), jnp.float32),
        mesh=mesh,
        scratch_types=[
            pltpu.VMEM_SHARED((N, D), jnp.float32),
            pltpu.VMEM((GC,), jnp.int32),
            pltpu.VMEM((GC,), jnp.int32),
            pltpu.VMEM((GC, D), jnp.float32),
            pltpu.VMEM((GC, D), jnp.float32),
            pltpu.SemaphoreType.DMA,
            pltpu.SemaphoreType.DMA,
            pltpu.SemaphoreType.DMA,
            pltpu.SemaphoreType.DMA,
            pltpu.SemaphoreType.DMA,
        ],
    )


@functools.lru_cache(maxsize=None)
def _make_scatter(nedge):
    ept = nedge // NS
    nout = ept // OC
    assert ept % OC == 0

    def body(msg_hbm, dst4d_hbm, prev_hbm, aggr_hbm, acc_sh,
             idx_all, rows0, rows1, isem, psem, vsem0, vsem1, ssem):
        c = lax.axis_index("c")
        s = lax.axis_index("s")
        tb = s * ept

        pltpu.async_copy(dst4d_hbm.at[s], idx_all, isem)

        for k in range(2):
            col0 = (c * 2 + k) * 128

            def msg_slice(o):
                return msg_hbm.at[pl.ds(tb + o * OC, OC), pl.ds(col0, 128)]

            def wait_load(o, rbuf, vsem):
                pltpu.make_async_copy(msg_slice(o), rbuf, vsem).wait()

            def scat(o, buf):
                ds_ = []
                for j in range(NSUB):
                    ds_.append(pltpu.async_copy(
                        buf.at[pl.ds(j * GC, GC)],
                        acc_sh.at[idx_all.at[o, j]], ssem, add=True))
                for d in ds_:
                    d.wait()

            pltpu.async_copy(
                prev_hbm.at[pl.ds(s * RPT, RPT), pl.ds(col0, 128)],
                acc_sh.at[pl.ds(s * RPT, RPT)], psem)
            @pl.when(s == NS - 1)
            def _():
                pltpu.async_copy(
                    prev_hbm.at[pl.ds(NS * RPT, TAIL), pl.ds(col0, 128)],
                    acc_sh.at[pl.ds(NS * RPT, TAIL)], psem)
            pltpu.async_copy(msg_slice(0), rows0, vsem0)
            pltpu.make_async_copy(
                prev_hbm.at[pl.ds(s * RPT, RPT), pl.ds(col0, 128)],
                acc_sh.at[pl.ds(s * RPT, RPT)], psem).wait()
            @pl.when(s == NS - 1)
            def _():
                pltpu.make_async_copy(
                    prev_hbm.at[pl.ds(NS * RPT, TAIL), pl.ds(col0, 128)],
                    acc_sh.at[pl.ds(NS * RPT, TAIL)], psem).wait()
            if k == 0:
                pltpu.make_async_copy(dst4d_hbm.at[s], idx_all, isem).wait()
            plsc.subcore_barrier()

            def pair(i, carry):
                o = i * 2
                pltpu.async_copy(msg_slice(o + 1), rows1, vsem1)
                wait_load(o, rows0, vsem0)
                scat(o, rows0)
                @pl.when(o + 2 < nout)
                def _():
                    pltpu.async_copy(msg_slice(o + 2), rows0, vsem0)
                wait_load(o + 1, rows1, vsem1)
                scat(o + 1, rows1)
                return carry

            lax.fori_loop(0, nout // 2, pair, 0)
            if nout % 2:
                wait_load(nout - 1, rows0, vsem0)
                scat(nout - 1, rows0)
            plsc.subcore_barrier()
            pltpu.sync_copy(acc_sh.at[pl.ds(s * RPT, RPT)],
                            aggr_hbm.at[pl.ds(s * RPT, RPT),
                                        pl.ds(col0, 128)])
            @pl.when(s == NS - 1)
            def _():
                pltpu.sync_copy(acc_sh.at[pl.ds(NS * RPT, TAIL)],
                                aggr_hbm.at[pl.ds(NS * RPT, TAIL),
                                            pl.ds(col0, 128)])

    mesh = plsc.VectorSubcoreMesh(core_axis_name="c", subcore_axis_name="s",
                                  num_cores=NC, num_subcores=NS)
    kern = pl.kernel(
        body,
        out_type=jax.ShapeDtypeStruct((N, 512), jnp.float32),
        mesh=mesh,
        scratch_types=[
            pltpu.VMEM_SHARED((N, 128), jnp.float32),
            pltpu.VMEM((nout, NSUB, GC), jnp.int32),
            pltpu.VMEM((OC, 128), jnp.float32),
            pltpu.VMEM((OC, 128), jnp.float32),
            pltpu.SemaphoreType.DMA,
            pltpu.SemaphoreType.DMA,
            pltpu.SemaphoreType.DMA,
            pltpu.SemaphoreType.DMA,
            pltpu.SemaphoreType.DMA,
        ],
    )

    def run(msg_seg, dst_seg, prev):
        dst4d = dst_seg.reshape(NS, nout, NSUB, GC)
        return kern(msg_seg, dst4d, prev)

    return run


def _edge_mlp_body(xj, ea, W1, b1, W2, b2, out):
    t = xj[...] * ea[...]
    h = _leaky(jnp.dot(t, W1[...], preferred_element_type=jnp.float32)
               + b1[...])
    out[...] = _leaky(jnp.dot(h, W2[...], preferred_element_type=jnp.float32)
                      + b2[...])


def _edge_mlp(xj, xj_off, ea, ea_off, nedge, W1, b1, W2, b2):
    BE = 800
    grid = (nedge // BE,)
    xo = xj_off // BE
    eo = ea_off // BE
    return pl.pallas_call(
        _edge_mlp_body,
        grid=grid,
        in_specs=[
            pl.BlockSpec((BE, D), lambda i: (xo + i, 0)),
            pl.BlockSpec((BE, D), lambda i: (eo + i, 0)),
            pl.BlockSpec((D, 256), lambda i: (0, 0)),
            pl.BlockSpec((1, 256), lambda i: (0, 0)),
            pl.BlockSpec((256, 512), lambda i: (0, 0)),
            pl.BlockSpec((1, 512), lambda i: (0, 0)),
        ],
        out_specs=pl.BlockSpec((BE, 512), lambda i: (i, 0)),
        out_shape=jax.ShapeDtypeStruct((nedge, 512), jnp.float32),
    )(xj, ea, W1, b1, W2, b2)


def _node_mlp_body(x0, aggr, W3a, W3b, b3, W4, b4s, out):
    bf = jnp.bfloat16
    u = _leaky(jnp.dot(x0[...].astype(bf), W3a[...].astype(bf),
                       preferred_element_type=jnp.float32)
               + jnp.dot(aggr[...].astype(bf), W3b[...].astype(bf),
                         preferred_element_type=jnp.float32)
               + b3[...])
    y = jnp.dot(u.astype(bf), W4[...].astype(bf),
                preferred_element_type=jnp.float32)
    out[...] = (y + b4s[...] + x0[...]) * 0.5


def _node_mlp(x0, aggr, W3a_s, W3b_s, b3, W4_s, b4s):
    BN = 1000
    grid = (N // BN,)
    return pl.pallas_call(
        _node_mlp_body,
        grid=grid,
        in_specs=[
            pl.BlockSpec((BN, D), lambda i: (i, 0)),
            pl.BlockSpec((BN, 512), lambda i: (i, 0)),
            pl.BlockSpec((D, 1024), lambda i: (0, 0)),
            pl.BlockSpec((512, 1024), lambda i: (0, 0)),
            pl.BlockSpec((1, 1024), lambda i: (0, 0)),
            pl.BlockSpec((1024, D), lambda i: (0, 0)),
            pl.BlockSpec((1, D), lambda i: (0, 0)),
        ],
        out_specs=pl.BlockSpec((BN, D), lambda i: (i, 0)),
        out_shape=jax.ShapeDtypeStruct((N, D), jnp.float32),
    )(x0, aggr, W3a_s, W3b_s, b3, W4_s, b4s)


def kernel(x0, edge_index, edge_attr, W1, b1, W2, b2, W3, b3, W4, b4,
           bn_g, bn_b, bn_m, bn_v):
    src = edge_index[0].astype(jnp.int32)
    dst = edge_index[1].astype(jnp.int32)

    b1r = b1.reshape(1, 256)
    b2r = b2.reshape(1, 512)

    xj_a = _make_gather(G0_SIZE)(x0, lax.slice(src, (0,), (G0_SIZE,)))
    xj_b = _make_gather(E - G0_SIZE)(x0, lax.slice(src, (G0_SIZE,), (E,)))

    aggr = jnp.zeros((N, 512), jnp.float32)
    lo = 0
    for sg, sz in enumerate(SEG_SIZES):
        if sg == 0:
            msg = _edge_mlp(xj_a, 0, edge_attr, 0, sz, W1, b1r, W2, b2r)
        else:
            msg = _edge_mlp(xj_b, lo - G0_SIZE, edge_attr, lo, sz,
                            W1, b1r, W2, b2r)
        aggr = _make_scatter(sz)(
            msg, lax.slice(dst, (lo,), (lo + sz,)), aggr)
        lo += sz

    scale = bn_g / jnp.sqrt(bn_v + 1e-5)
    shift = (b4 - bn_m) * scale + bn_b
    W4_s = W4 * scale[None, :]
    b4s = shift.reshape(1, D)
    out = _node_mlp(x0, aggr, W3[:D], W3[D:], b3.reshape(1, 1024), W4_s, b4s)
    return out


# speedup vs baseline: 1.0648x; 1.0012x over previous
import functools

import jax
import jax.numpy as jnp
from jax import lax
from jax.experimental import pallas as pl
from jax.experimental.pallas import tpu as pltpu
from jax.experimental.pallas import tpu_sc as plsc

N = 10000
E = 320000
D = 128

NC = 2
NS = 16
NW = NC * NS

GC = 80
OC = 160
NSUB = OC // GC

SEG_SIZES = (51200, 89600, 89600, 89600)
G0_SIZE = SEG_SIZES[0]

RPT = 624
TAIL = N - NS * RPT


def _leaky(z):
    return jnp.where(z > 0, z, 0.01 * z)


@functools.lru_cache(maxsize=None)
def _make_gather(nedge):
    epw = nedge // NW
    ng = epw // GC
    assert epw % GC == 0

    def body(x0_hbm, src_hbm, xj_hbm, x0_sh, idx0, idx1, rows0, rows1,
             isem0, isem1, gsem, wsem0, wsem1):
        c = lax.axis_index("c")
        s = lax.axis_index("s")
        wid = s * NC + c
        wb = wid * epw
        pltpu.sync_copy(x0_hbm.at[pl.ds(s * RPT, RPT)],
                        x0_sh.at[pl.ds(s * RPT, RPT)])
        @pl.when(s == NS - 1)
        def _():
            pltpu.sync_copy(x0_hbm.at[pl.ds(NS * RPT, TAIL)],
                            x0_sh.at[pl.ds(NS * RPT, TAIL)])
        plsc.subcore_barrier()

        def src_slice(o):
            return src_hbm.at[pl.ds(pl.multiple_of(wb + o * GC, GC), GC)]

        def xj_slice(o):
            return xj_hbm.at[pl.ds(pl.multiple_of(wb + o * GC, GC), GC)]

        def stage(o, ibuf, rbuf, isem, wsem, first):
            pltpu.make_async_copy(src_slice(o), ibuf, isem).wait()
            @pl.when(jnp.logical_not(first))
            def _():
                pltpu.make_async_copy(rbuf, xj_slice(o - 2), wsem).wait()
            pltpu.async_copy(x0_sh.at[ibuf], rbuf, gsem).wait()
            pltpu.async_copy(rbuf, xj_slice(o), wsem)

        pltpu.async_copy(src_slice(0), idx0, isem0)

        def pair(i, carry):
            o = i * 2
            pltpu.async_copy(src_slice(o + 1), idx1, isem1)
            stage(o, idx0, rows0, isem0, wsem0, i == 0)
            @pl.when(o + 2 < ng)
            def _():
                pltpu.async_copy(src_slice(o + 2), idx0, isem0)
            stage(o + 1, idx1, rows1, isem1, wsem1, i == 0)
            return carry

        lax.fori_loop(0, ng // 2, pair, 0)
        if ng % 2:
            stage(ng - 1, idx0, rows0, isem0, wsem0, False)
            pltpu.make_async_copy(rows1, xj_slice(ng - 2), wsem1).wait()
            pltpu.make_async_copy(rows0, xj_slice(ng - 1), wsem0).wait()
        else:
            pltpu.make_async_copy(rows0, xj_slice(ng - 2), wsem0).wait()
            pltpu.make_async_copy(rows1, xj_slice(ng - 1), wsem1).wait()

    mesh = plsc.VectorSubcoreMesh(core_axis_name="c", subcore_axis_name="s",
                                  num_cores=NC, num_subcores=NS)
    return pl.kernel(
        body,
        out_type=jax.ShapeDtypeStruct((nedge, D), jnp.float32),
        mesh=mesh,
        scratch_types=[
            pltpu.VMEM_SHARED((N, D), jnp.float32),
            pltpu.VMEM((GC,), jnp.int32),
            pltpu.VMEM((GC,), jnp.int32),
            pltpu.VMEM((GC, D), jnp.float32),
            pltpu.VMEM((GC, D), jnp.float32),
            pltpu.SemaphoreType.DMA,
            pltpu.SemaphoreType.DMA,
            pltpu.SemaphoreType.DMA,
            pltpu.SemaphoreType.DMA,
            pltpu.SemaphoreType.DMA,
        ],
    )


@functools.lru_cache(maxsize=None)
def _make_scatter(nedge):
    ept = nedge // NS
    nout = ept // OC
    assert ept % OC == 0

    def body(msg_hbm, dst4d_hbm, prev_hbm, aggr_hbm, acc_sh,
             idx_all, rows0, rows1, isem, psem, vsem0, vsem1, ssem):
        c = lax.axis_index("c")
        s = lax.axis_index("s")
        tb = s * ept

        pltpu.async_copy(dst4d_hbm.at[s], idx_all, isem)

        for k in range(2):
            col0 = (c * 2 + k) * 128

            def msg_slice(o):
                return msg_hbm.at[pl.ds(tb + o * OC, OC), pl.ds(col0, 128)]

            def wait_load(o, rbuf, vsem):
                pltpu.make_async_copy(msg_slice(o), rbuf, vsem).wait()

            def scat(o, buf):
                ds_ = []
                for j in range(NSUB):
                    ds_.append(pltpu.async_copy(
                        buf.at[pl.ds(j * GC, GC)],
                        acc_sh.at[idx_all.at[o, j]], ssem, add=True))
                for d in ds_:
                    d.wait()

            pltpu.async_copy(
                prev_hbm.at[pl.ds(s * RPT, RPT), pl.ds(col0, 128)],
                acc_sh.at[pl.ds(s * RPT, RPT)], psem)
            @pl.when(s == NS - 1)
            def _():
                pltpu.async_copy(
                    prev_hbm.at[pl.ds(NS * RPT, TAIL), pl.ds(col0, 128)],
                    acc_sh.at[pl.ds(NS * RPT, TAIL)], psem)
            pltpu.async_copy(msg_slice(0), rows0, vsem0)
            pltpu.make_async_copy(
                prev_hbm.at[pl.ds(s * RPT, RPT), pl.ds(col0, 128)],
                acc_sh.at[pl.ds(s * RPT, RPT)], psem).wait()
            @pl.when(s == NS - 1)
            def _():
                pltpu.make_async_copy(
                    prev_hbm.at[pl.ds(NS * RPT, TAIL), pl.ds(col0, 128)],
                    acc_sh.at[pl.ds(NS * RPT, TAIL)], psem).wait()
            if k == 0:
                pltpu.make_async_copy(dst4d_hbm.at[s], idx_all, isem).wait()
            plsc.subcore_barrier()

            def pair(i, carry):
                o = i * 2
                pltpu.async_copy(msg_slice(o + 1), rows1, vsem1)
                wait_load(o, rows0, vsem0)
                scat(o, rows0)
                @pl.when(o + 2 < nout)
                def _():
                    pltpu.async_copy(msg_slice(o + 2), rows0, vsem0)
                wait_load(o + 1, rows1, vsem1)
                scat(o + 1, rows1)
                return carry

            lax.fori_loop(0, nout // 2, pair, 0)
            if nout % 2:
                wait_load(nout - 1, rows0, vsem0)
                scat(nout - 1, rows0)
            plsc.subcore_barrier()
            pltpu.sync_copy(acc_sh.at[pl.ds(s * RPT, RPT)],
                            aggr_hbm.at[pl.ds(s * RPT, RPT),
                                        pl.ds(col0, 128)])
            @pl.when(s == NS - 1)
            def _():
                pltpu.sync_copy(acc_sh.at[pl.ds(NS * RPT, TAIL)],
                                aggr_hbm.at[pl.ds(NS * RPT, TAIL),
                                            pl.ds(col0, 128)])

    mesh = plsc.VectorSubcoreMesh(core_axis_name="c", subcore_axis_name="s",
                                  num_cores=NC, num_subcores=NS)
    kern = pl.kernel(
        body,
        out_type=jax.ShapeDtypeStruct((N, 512), jnp.float32),
        mesh=mesh,
        scratch_types=[
            pltpu.VMEM_SHARED((N, 128), jnp.float32),
            pltpu.VMEM((nout, NSUB, GC), jnp.int32),
            pltpu.VMEM((OC, 128), jnp.float32),
            pltpu.VMEM((OC, 128), jnp.float32),
            pltpu.SemaphoreType.DMA,
            pltpu.SemaphoreType.DMA,
            pltpu.SemaphoreType.DMA,
            pltpu.SemaphoreType.DMA,
            pltpu.SemaphoreType.DMA,
        ],
    )

    def run(msg_seg, dst_seg, prev):
        dst4d = dst_seg.reshape(NS, nout, NSUB, GC)
        return kern(msg_seg, dst4d, prev)

    return run


def _edge_mlp_body(xj, ea, W1, b1, W2, b2, out):
    t = xj[...] * ea[...]
    h = _leaky(jnp.dot(t, W1[...], preferred_element_type=jnp.float32)
               + b1[...])
    out[...] = _leaky(jnp.dot(h, W2[...], preferred_element_type=jnp.float32)
                      + b2[...])


def _edge_mlp(xj, xj_off, ea, ea_off, nedge, W1, b1, W2, b2):
    BE = 800
    grid = (nedge // BE,)
    xo = xj_off // BE
    eo = ea_off // BE
    return pl.pallas_call(
        _edge_mlp_body,
        grid=grid,
        in_specs=[
            pl.BlockSpec((BE, D), lambda i: (xo + i, 0)),
            pl.BlockSpec((BE, D), lambda i: (eo + i, 0)),
            pl.BlockSpec((D, 256), lambda i: (0, 0)),
            pl.BlockSpec((1, 256), lambda i: (0, 0)),
            pl.BlockSpec((256, 512), lambda i: (0, 0)),
            pl.BlockSpec((1, 512), lambda i: (0, 0)),
        ],
        out_specs=pl.BlockSpec((BE, 512), lambda i: (i, 0)),
        out_shape=jax.ShapeDtypeStruct((nedge, 512), jnp.float32),
    )(xj, ea, W1, b1, W2, b2)


def _node_mlp_body(x0, aggr, W3a, W3b, b3, W4, b4s, out):
    bf = jnp.bfloat16
    u = _leaky(jnp.dot(x0[...].astype(bf), W3a[...].astype(bf),
                       preferred_element_type=jnp.float32)
               + jnp.dot(aggr[...].astype(bf), W3b[...].astype(bf),
                         preferred_element_type=jnp.float32)
               + b3[...])
    y = jnp.dot(u.astype(bf), W4[...].astype(bf),
                preferred_element_type=jnp.float32)
    out[...] = (y + b4s[...] + x0[...]) * 0.5


def _node_mlp(x0, aggr, W3a_s, W3b_s, b3, W4_s, b4s):
    BN = 2000
    grid = (N // BN,)
    return pl.pallas_call(
        _node_mlp_body,
        grid=grid,
        in_specs=[
            pl.BlockSpec((BN, D), lambda i: (i, 0)),
            pl.BlockSpec((BN, 512), lambda i: (i, 0)),
            pl.BlockSpec((D, 1024), lambda i: (0, 0)),
            pl.BlockSpec((512, 1024), lambda i: (0, 0)),
            pl.BlockSpec((1, 1024), lambda i: (0, 0)),
            pl.BlockSpec((1024, D), lambda i: (0, 0)),
            pl.BlockSpec((1, D), lambda i: (0, 0)),
        ],
        out_specs=pl.BlockSpec((BN, D), lambda i: (i, 0)),
        out_shape=jax.ShapeDtypeStruct((N, D), jnp.float32),
    )(x0, aggr, W3a_s, W3b_s, b3, W4_s, b4s)


def kernel(x0, edge_index, edge_attr, W1, b1, W2, b2, W3, b3, W4, b4,
           bn_g, bn_b, bn_m, bn_v):
    src = edge_index[0].astype(jnp.int32)
    dst = edge_index[1].astype(jnp.int32)

    b1r = b1.reshape(1, 256)
    b2r = b2.reshape(1, 512)

    xj_a = _make_gather(G0_SIZE)(x0, lax.slice(src, (0,), (G0_SIZE,)))
    xj_b = _make_gather(E - G0_SIZE)(x0, lax.slice(src, (G0_SIZE,), (E,)))

    aggr = jnp.zeros((N, 512), jnp.float32)
    lo = 0
    for sg, sz in enumerate(SEG_SIZES):
        if sg == 0:
            msg = _edge_mlp(xj_a, 0, edge_attr, 0, sz, W1, b1r, W2, b2r)
        else:
            msg = _edge_mlp(xj_b, lo - G0_SIZE, edge_attr, lo, sz,
                            W1, b1r, W2, b2r)
        aggr = _make_scatter(sz)(
            msg, lax.slice(dst, (lo,), (lo + sz,)), aggr)
        lo += sz

    scale = bn_g / jnp.sqrt(bn_v + 1e-5)
    shift = (b4 - bn_m) * scale + bn_b
    W4_s = W4 * scale[None, :]
    b4s = shift.reshape(1, D)
    out = _node_mlp(x0, aggr, W3[:D], W3[D:], b3.reshape(1, 1024), W4_s, b4s)
    return out
